# baseline (speedup 1.0000x reference)
import functools

import jax
import jax.numpy as jnp
from jax import lax
from jax.experimental import pallas as pl
from jax.experimental.pallas import tpu as pltpu
from jax.experimental.pallas import tpu_sc as plsc

NC = 2
NS = 16
L = 16
NW = NC * NS

BATCH = 16384
D = 32
V = 1_000_000
N_SC = 8192
K_TC = 8


def _gmf_sc(uid, iid, ut_t, it_t, nb):
    b_per_w = nb // NW
    mesh = plsc.VectorSubcoreMesh(core_axis_name="c", subcore_axis_name="s")

    @functools.partial(
        pl.kernel,
        mesh=mesh,
        out_type=jax.ShapeDtypeStruct((D, nb), jnp.float32),
        compiler_params=pltpu.CompilerParams(needs_layout_passes=False),
        scratch_types=[
            pltpu.VMEM((b_per_w,), jnp.int32),
            pltpu.VMEM((b_per_w,), jnp.int32),
            pltpu.VMEM((3 * 4, D, 128), jnp.float32),
            pltpu.VMEM((3 * 4, D, 128), jnp.float32),
            pltpu.VMEM((D, b_per_w), jnp.float32),
            pltpu.SemaphoreType.DMA,
            pltpu.SemaphoreType.DMA,
        ],
    )
    def k(uid_hbm, iid_hbm, ut_hbm, it_hbm, out_hbm,
          uloc, iloc, ublk, iblk, obuf, usem, isem):
        wid = lax.axis_index("s") * NC + lax.axis_index("c")
        b0 = wid * b_per_w

        pltpu.sync_copy(uid_hbm.at[pl.ds(b0, b_per_w)], uloc)
        pltpu.sync_copy(iid_hbm.at[pl.ds(b0, b_per_w)], iloc)

        rows_lo = lax.iota(jnp.int32, L)
        rows_hi = rows_lo + L

        SB = 4

        def step(g, carry):
            vu = uloc[pl.ds(g * L, L)]
            vi = iloc[pl.ds(g * L, L)]
            tu = vu >> 7
            wu = vu & 127
            ti = vi >> 7
            wi = vi & 127
            cps = {}

            def fire(h):
                lst = []
                for j in range(SB):
                    jj = h * SB + j
                    slot = (h % 3) * SB + j
                    lst.append(pltpu.async_copy(
                        ut_hbm.at[:, pl.ds(tu[jj] * 128, 128)],
                        ublk.at[slot], usem))
                    lst.append(pltpu.async_copy(
                        it_hbm.at[:, pl.ds(ti[jj] * 128, 128)],
                        iblk.at[slot], isem))
                cps[h] = lst

            fire(0)
            fire(1)
            for h in range(L // SB):
                if h < L // SB - 2:
                    fire(h + 2)
                for cp in cps[h]:
                    cp.wait()
                for j in range(SB):
                    jj = h * SB + j
                    slot = (h % 3) * SB + j
                    b = g * L + jj
                    bcol = jnp.full((L,), b, jnp.int32)
                    wub = jnp.full((L,), wu[jj], jnp.int32)
                    wib = jnp.full((L,), wi[jj], jnp.int32)
                    for rows in (rows_lo, rows_hi):
                        uv = plsc.load_gather(ublk.at[slot], [rows, wub])
                        iv = plsc.load_gather(iblk.at[slot], [rows, wib])
                        plsc.store_scatter(obuf, [rows, bcol], uv * iv)
            return carry

        lax.fori_loop(0, b_per_w // L, step, 0)

        pltpu.sync_copy(obuf, out_hbm.at[:, pl.ds(b0, b_per_w)])

    return k(uid, iid, ut_t, it_t)


def _gmf_tc(uid, iid, ut_t, it_t, nb):
    grid = nb // K_TC
    tu = uid >> 7
    wu = uid & 127
    ti = iid >> 7
    wi = iid & 127

    def body(tu_ref, ti_ref, wu_ref, wi_ref, *refs):
        ublks = refs[:K_TC]
        iblks = refs[K_TC:2 * K_TC]
        out = refs[2 * K_TC]
        i = pl.program_id(0)
        lane = lax.broadcasted_iota(jnp.int32, (D, 128), 1)
        parts = []
        for j in range(K_TC):
            w_u = wu_ref[i * K_TC + j]
            w_i = wi_ref[i * K_TC + j]
            su = jnp.sum(jnp.where(lane == w_u, ublks[j][...], 0.0), axis=1)
            si = jnp.sum(jnp.where(lane == w_i, iblks[j][...], 0.0), axis=1)
            parts.append((su * si).reshape(1, D))
        out[0] = jnp.concatenate(parts, axis=0)

    def u_map(j):
        return lambda i, tu_r, ti_r, wu_r, wi_r: (0, tu_r[i * K_TC + j])

    def i_map(j):
        return lambda i, tu_r, ti_r, wu_r, wi_r: (0, ti_r[i * K_TC + j])

    grid_spec = pltpu.PrefetchScalarGridSpec(
        num_scalar_prefetch=4,
        grid=(grid,),
        in_specs=(
            [pl.BlockSpec((D, 128), u_map(j)) for j in range(K_TC)]
            + [pl.BlockSpec((D, 128), i_map(j)) for j in range(K_TC)]
        ),
        out_specs=pl.BlockSpec(
            (1, K_TC, D), lambda i, *_: (i, 0, 0)),
    )
    out3 = pl.pallas_call(
        body,
        grid_spec=grid_spec,
        out_shape=jax.ShapeDtypeStruct((grid, K_TC, D), jnp.float32),
    )(tu, ti, wu, wi, *([ut_t] * K_TC + [it_t] * K_TC))
    return out3.reshape(nb, D)


def kernel(user_ids, item_ids, user_table, item_table):
    uid = user_ids.astype(jnp.int32)
    iid = item_ids.astype(jnp.int32)
    ut_t = user_table.T
    it_t = item_table.T
    out_sc = _gmf_sc(uid[:N_SC], iid[:N_SC], ut_t, it_t, N_SC)
    out_tc = _gmf_tc(uid[N_SC:], iid[N_SC:], ut_t, it_t, BATCH - N_SC)
    return jnp.concatenate([out_sc.T, out_tc], axis=0)


# speedup vs baseline: 3.0193x; 3.0193x over previous
import functools

import jax
import jax.numpy as jnp
from jax import lax
from jax.experimental import pallas as pl
from jax.experimental.pallas import tpu as pltpu
from jax.experimental.pallas import tpu_sc as plsc

NC = 2
NS = 16
L = 16
NW = NC * NS

BATCH = 16384
D = 32
V = 1_000_000
B_PER_W = BATCH // NW


def _gmf_sc(uid, iid, ut_t, it_t):
    mesh = plsc.VectorSubcoreMesh(core_axis_name="c", subcore_axis_name="s")

    @functools.partial(
        pl.kernel,
        mesh=mesh,
        out_type=jax.ShapeDtypeStruct((D, BATCH), jnp.float32),
        compiler_params=pltpu.CompilerParams(needs_layout_passes=False),
        scratch_types=[
            pltpu.VMEM((B_PER_W,), jnp.int32),
            pltpu.VMEM((B_PER_W,), jnp.int32),
            pltpu.VMEM((3 * 4, D, 128), jnp.float32),
            pltpu.VMEM((3 * 4, D, 128), jnp.float32),
            pltpu.VMEM((D, B_PER_W), jnp.float32),
            pltpu.SemaphoreType.DMA,
            pltpu.SemaphoreType.DMA,
        ],
    )
    def k(uid_hbm, iid_hbm, ut_hbm, it_hbm, out_hbm,
          uloc, iloc, ublk, iblk, obuf, usem, isem):
        wid = lax.axis_index("s") * NC + lax.axis_index("c")
        b0 = wid * B_PER_W

        pltpu.sync_copy(uid_hbm.at[pl.ds(b0, B_PER_W)], uloc)
        pltpu.sync_copy(iid_hbm.at[pl.ds(b0, B_PER_W)], iloc)

        rows_lo = lax.iota(jnp.int32, L)
        rows_hi = rows_lo + L

        SB = 4

        def step(g, carry):
            vu = uloc[pl.ds(g * L, L)]
            vi = iloc[pl.ds(g * L, L)]
            tu = vu >> 7
            wu = vu & 127
            ti = vi >> 7
            wi = vi & 127
            cps = {}

            def fire(h):
                lst = []
                for j in range(SB):
                    jj = h * SB + j
                    slot = (h % 3) * SB + j
                    lst.append(pltpu.async_copy(
                        ut_hbm.at[:, pl.ds(tu[jj] * 128, 128)],
                        ublk.at[slot], usem))
                    lst.append(pltpu.async_copy(
                        it_hbm.at[:, pl.ds(ti[jj] * 128, 128)],
                        iblk.at[slot], isem))
                cps[h] = lst

            fire(0)
            fire(1)
            for h in range(L // SB):
                if h < L // SB - 2:
                    fire(h + 2)
                for cp in cps[h]:
                    cp.wait()
                for j in range(SB):
                    jj = h * SB + j
                    slot = (h % 3) * SB + j
                    b = g * L + jj
                    bcol = jnp.full((L,), b, jnp.int32)
                    wub = jnp.full((L,), wu[jj], jnp.int32)
                    wib = jnp.full((L,), wi[jj], jnp.int32)
                    for rows in (rows_lo, rows_hi):
                        uv = plsc.load_gather(ublk.at[slot], [rows, wub])
                        iv = plsc.load_gather(iblk.at[slot], [rows, wib])
                        plsc.store_scatter(obuf, [rows, bcol], uv * iv)
            return carry

        lax.fori_loop(0, B_PER_W // L, step, 0)

        pltpu.sync_copy(obuf, out_hbm.at[:, pl.ds(b0, B_PER_W)])

    return k(uid, iid, ut_t, it_t)


def kernel(user_ids, item_ids, user_table, item_table):
    uid = user_ids.astype(jnp.int32)
    iid = item_ids.astype(jnp.int32)
    out_t = _gmf_sc(uid, iid, user_table.T, item_table.T)
    return out_t.T
